# baseline (speedup 1.0000x reference)
import functools

import jax
import jax.numpy as jnp
from jax import lax
from jax.experimental import pallas as pl
from jax.experimental.pallas import tpu as pltpu
from jax.experimental.pallas import tpu_sc as plsc

N = 10000
E = 320000
NG = 64
ED = 128
DD = 32
CD = 32
H = 128
DD1 = 51
CD1 = 21

NC = 2
NS = 16
NW = NC * NS

EPT = E // NW
CH = 80
NCHUNK = EPT // CH
NODE_CHUNKS = N // CH
ZR = 80
ECH = 128
EPTP = 10240
ENCHUNK = EPTP // ECH
EP = NW * EPTP
NP = N + ZR
PCH = 8
NPAN = ENCHUNK // PCH
DEGW = 640
DEGP = NS * DEGW

_dot = functools.partial(
    jnp.dot, precision=lax.Precision.HIGHEST, preferred_element_type=jnp.float32
)


def _sc_embed_body(x_h, dst_h, ntab_h,
                   nf_h, deg0_h, deg1_h,
                   xv, dv, nfv, ones_v, zb, deg_sh, sem):
    c = lax.axis_index("c")
    s = lax.axis_index("s")
    w = c * NS + s

    def _fill_zb(i, carry):
        zb[pl.ds(i * 16, 16)] = jnp.zeros((16,), jnp.float32)
        return carry
    lax.fori_loop(0, DEGW // 16, _fill_zb, 0)

    def _fill_ones(i, carry):
        ones_v[pl.ds(i * 16, 16)] = jnp.ones((16,), jnp.float32)
        return carry
    lax.fori_loop(0, CH // 16, _fill_ones, 0)

    pltpu.sync_copy(zb, deg_sh.at[pl.ds(s * DEGW, DEGW)])
    plsc.subcore_barrier()

    def _deg(i, carry):
        base = w * EPT + i * CH
        pltpu.sync_copy(dst_h.at[pl.ds(base, CH)], dv)
        pltpu.sync_copy(ones_v, deg_sh.at[dv], add=True)
        return carry
    lax.fori_loop(0, NCHUNK, _deg, 0)

    def _emb(i, carry):
        j = w + NW * i

        @pl.when(j < NODE_CHUNKS)
        def _():
            base = j * CH
            pltpu.sync_copy(x_h.at[pl.ds(base, CH)], xv)
            pltpu.async_copy(ntab_h.at[xv], nfv, sem).wait()
            pltpu.sync_copy(nfv, nf_h.at[pl.ds(base, CH)])
        return carry
    lax.fori_loop(0, (NODE_CHUNKS + NW - 1) // NW, _emb, 0)

    plsc.subcore_barrier()

    @pl.when(c == 0)
    def _():
        pltpu.sync_copy(deg_sh.at[pl.ds(s * DEGW, DEGW)],
                        deg0_h.at[pl.ds(s * DEGW, DEGW)])

    @pl.when(c == 1)
    def _():
        pltpu.sync_copy(deg_sh.at[pl.ds(s * DEGW, DEGW)],
                        deg1_h.at[pl.ds(s * DEGW, DEGW)])


@functools.cache
def _get_sc_embed():
    return pl.kernel(
        _sc_embed_body,
        out_type=[
            jax.ShapeDtypeStruct((N, ED), jnp.float32),
            jax.ShapeDtypeStruct((DEGP,), jnp.float32),
            jax.ShapeDtypeStruct((DEGP,), jnp.float32),
        ],
        scratch_types=[
            pltpu.VMEM((CH,), jnp.int32),
            pltpu.VMEM((CH,), jnp.int32),
            pltpu.VMEM((CH, ED), jnp.float32),
            pltpu.VMEM((CH,), jnp.float32),
            pltpu.VMEM((DEGW,), jnp.float32),
            pltpu.VMEM_SHARED((DEGP,), jnp.float32),
            pltpu.SemaphoreType.DMA,
        ],
        mesh=plsc.VectorSubcoreMesh(core_axis_name="c", subcore_axis_name="s"),
    )


def _panel(hs_h, acc_sh, sv, dv, rows0, rows1, gsem0, gsem1):
    pltpu.async_copy(hs_h.at[sv.at[0]], rows0, gsem0)
    for j in range(PCH // 2):
        c0 = 2 * j
        c1 = c0 + 1
        pltpu.async_copy(hs_h.at[sv.at[c1]], rows1, gsem1)
        pltpu.make_async_copy(hs_h.at[sv.at[c0]], rows0, gsem0).wait()
        pltpu.sync_copy(rows0, acc_sh.at[dv.at[c0]], add=True)
        if c1 + 1 < PCH:
            pltpu.async_copy(hs_h.at[sv.at[c1 + 1]], rows0, gsem0)
        pltpu.make_async_copy(hs_h.at[sv.at[c1]], rows1, gsem1).wait()
        pltpu.sync_copy(rows1, acc_sh.at[dv.at[c1]], add=True)


def _sc_scatter_body(src_h, dst_h, hs_h, out_h,
                     svA, dvA, svB, dvB, rows0, rows1, acc_sh,
                     gsem0, gsem1, isemA, isemB):
    c = lax.axis_index("c")
    s = lax.axis_index("s")
    w = c * NS + s
    base = w * ENCHUNK

    def _zrow(r, carry):
        def _zcol(k, inner):
            rows0[r, pl.ds(k * 16, 16)] = jnp.zeros((16,), jnp.float32)
            return inner
        return lax.fori_loop(0, H // 16, _zcol, carry)
    lax.fori_loop(0, ECH, _zrow, 0)

    def _zacc(t, carry):
        j = s + NS * t

        @pl.when(j < NP // ZR)
        def _():
            pltpu.sync_copy(rows0.at[pl.ds(0, ZR)], acc_sh.at[pl.ds(j * ZR, ZR)])
        return carry
    lax.fori_loop(0, (NP // ZR + NS - 1) // NS, _zacc, 0)
    plsc.subcore_barrier()

    pltpu.sync_copy(src_h.at[pl.ds(base, PCH)], svA)
    pltpu.sync_copy(dst_h.at[pl.ds(base, PCH)], dvA)

    def _pp(pp, carry):
        pa = base + 2 * pp * PCH
        pltpu.async_copy(src_h.at[pl.ds(pa + PCH, PCH)], svB, isemB)
        pltpu.async_copy(dst_h.at[pl.ds(pa + PCH, PCH)], dvB, isemB)
        _panel(hs_h, acc_sh, svA, dvA, rows0, rows1, gsem0, gsem1)
        pltpu.make_async_copy(src_h.at[pl.ds(pa + PCH, PCH)], svB, isemB).wait()
        pltpu.make_async_copy(dst_h.at[pl.ds(pa + PCH, PCH)], dvB, isemB).wait()

        @pl.when(pp < NPAN // 2 - 1)
        def _():
            pltpu.async_copy(src_h.at[pl.ds(pa + 2 * PCH, PCH)], svA, isemA)
            pltpu.async_copy(dst_h.at[pl.ds(pa + 2 * PCH, PCH)], dvA, isemA)
        _panel(hs_h, acc_sh, svB, dvB, rows0, rows1, gsem0, gsem1)

        @pl.when(pp < NPAN // 2 - 1)
        def _():
            pltpu.make_async_copy(src_h.at[pl.ds(pa + 2 * PCH, PCH)], svA,
                                  isemA).wait()
            pltpu.make_async_copy(dst_h.at[pl.ds(pa + 2 * PCH, PCH)], dvA,
                                  isemA).wait()
        return carry
    lax.fori_loop(0, NPAN // 2, _pp, 0)

    plsc.subcore_barrier()

    def _drain(t, carry):
        j = s + NS * t

        @pl.when(j < N // ZR)
        def _():
            pltpu.sync_copy(acc_sh.at[pl.ds(j * ZR, ZR)],
                            out_h.at[c, pl.ds(j * ZR, ZR)])
        return carry
    lax.fori_loop(0, (N // ZR + NS - 1) // NS, _drain, 0)


@functools.cache
def _get_sc_scatter():
    return pl.kernel(
        _sc_scatter_body,
        out_type=[jax.ShapeDtypeStruct((NC, N, H), jnp.float32)],
        scratch_types=[
            pltpu.VMEM((PCH, ECH), jnp.int32),
            pltpu.VMEM((PCH, ECH), jnp.int32),
            pltpu.VMEM((PCH, ECH), jnp.int32),
            pltpu.VMEM((PCH, ECH), jnp.int32),
            pltpu.VMEM((ECH, H), jnp.float32),
            pltpu.VMEM((ECH, H), jnp.float32),
            pltpu.VMEM_SHARED((NP, H), jnp.float32),
            pltpu.SemaphoreType.DMA,
            pltpu.SemaphoreType.DMA,
            pltpu.SemaphoreType.DMA,
            pltpu.SemaphoreType.DMA,
        ],
        mesh=plsc.VectorSubcoreMesh(core_axis_name="c", subcore_axis_name="s"),
    )


R = 1000


def _tc_proj_body(nf, ndb, cib, degp, dtab, ctab, wn, wd, wc, pb, w1, hs_out):
    dw = _dot(dtab[...], wd[...])
    cw = _dot(ctab[...], wc[...])
    oh_d = (lax.broadcasted_iota(jnp.int32, (R, DD1), 1) == ndb[...]).astype(
        jnp.float32)
    oh_c = (lax.broadcasted_iota(jnp.int32, (R, CD1), 1) == cib[...]).astype(
        jnp.float32)
    h = _dot(nf[...], wn[...]) + _dot(oh_d, dw) + _dot(oh_c, cw)
    h = jnp.maximum(h + pb[...], 0.0)
    dinv = lax.rsqrt(1.0 + degp[:, 0] + degp[:, 1])
    hs_out[...] = _dot(h, w1[...]) * dinv[:, None]


def _tc_mid_body(p0, p1, hs1, degp, b1, w2, hs2_out):
    dinv = lax.rsqrt(1.0 + degp[:, 0] + degp[:, 1])[:, None]
    h2 = jnp.maximum((p0[...] + p1[...] + hs1[...]) * dinv + b1[...], 0.0)
    hs2_out[...] = _dot(h2, w2[...]) * dinv


def _tc_final_body(q0, q1, hs2, degp, b2, batch, cw, cb, out):
    dinv = lax.rsqrt(1.0 + degp[:, 0] + degp[:, 1])[:, None]
    h3 = jnp.maximum((q0[...] + q1[...] + hs2[...]) * dinv + b2[...], 0.0)
    seg = lax.broadcasted_iota(jnp.int32, (NG, N), 0)
    m = (seg == batch[...]).astype(jnp.float32)
    sums = _dot(m, h3)
    counts = jnp.sum(m, axis=1, keepdims=True)
    pooled = sums / jnp.maximum(counts, 1.0)
    out[...] = _dot(pooled, cw[...]) + cb[...]


def _row_spec(cols):
    return pl.BlockSpec((R, cols), lambda j: (j, 0))


def _bcast_spec(rows, cols):
    return pl.BlockSpec((rows, cols), lambda j: (0, 0))


_deg_spec = pl.BlockSpec((R, NC), lambda j: (j, 0))

_tc_proj = pl.pallas_call(
    _tc_proj_body,
    grid=(N // R,),
    in_specs=[
        _row_spec(ED), _row_spec(1), _row_spec(1), _deg_spec,
        _bcast_spec(DD1, DD), _bcast_spec(CD1, CD),
        _bcast_spec(ED, H), _bcast_spec(DD, H), _bcast_spec(CD, H),
        _bcast_spec(1, H), _bcast_spec(H, H),
    ],
    out_specs=_row_spec(H),
    out_shape=jax.ShapeDtypeStruct((N, H), jnp.float32),
)

_tc_mid = pl.pallas_call(
    _tc_mid_body,
    grid=(N // R,),
    in_specs=[
        _row_spec(H), _row_spec(H), _row_spec(H), _deg_spec,
        _bcast_spec(1, H), _bcast_spec(H, H),
    ],
    out_specs=_row_spec(H),
    out_shape=jax.ShapeDtypeStruct((N, H), jnp.float32),
)

_tc_final = pl.pallas_call(
    _tc_final_body,
    out_shape=jax.ShapeDtypeStruct((NG, 1), jnp.float32),
)


def kernel(x, edge_index, batch, node_depth, child_index, node_table,
           depth_table, child_table, proj_W, proj_b, conv1_W, conv1_b,
           conv2_W, conv2_b, clf_W, clf_b):
    src = edge_index[0]
    dst = edge_index[1]

    nf, deg0, deg1 = _get_sc_embed()(
        x.astype(jnp.int32), dst.astype(jnp.int32), node_table)
    degp = jnp.stack([deg0[:N], deg1[:N]], axis=1)

    wn = proj_W[:ED]
    wd = proj_W[ED:ED + DD]
    wc = proj_W[ED + DD:]

    hs1 = _tc_proj(nf, node_depth.astype(jnp.int32)[:, None],
                   child_index.astype(jnp.int32)[:, None], degp,
                   depth_table, child_table, wn, wd, wc, proj_b[None, :],
                   conv1_W)
    pad = EP - E
    srcp = jnp.concatenate(
        [src.astype(jnp.int32), jnp.zeros((pad,), jnp.int32)]
    ).reshape(NW * ENCHUNK, ECH)
    dstp = jnp.concatenate(
        [dst.astype(jnp.int32), jnp.full((pad,), N, jnp.int32)]
    ).reshape(NW * ENCHUNK, ECH)
    scatter = _get_sc_scatter()
    p = scatter(srcp, dstp, hs1)[0]
    hs2 = _tc_mid(p[0], p[1], hs1, degp, conv1_b[None, :], conv2_W)
    q = scatter(srcp, dstp, hs2)[0]
    return _tc_final(q[0], q[1], hs2, degp, conv2_b[None, :],
                     batch[None, :].astype(jnp.int32), clf_W, clf_b[None, :])


# speedup vs baseline: 1.0492x; 1.0492x over previous
import functools

import jax
import jax.numpy as jnp
from jax import lax
from jax.experimental import pallas as pl
from jax.experimental.pallas import tpu as pltpu
from jax.experimental.pallas import tpu_sc as plsc

N = 10000
E = 320000
NG = 64
ED = 128
DD = 32
CD = 32
H = 128
DD1 = 51
CD1 = 21

NC = 2
NS = 16
NW = NC * NS

EPT = E // NW
CH = 80
NCHUNK = EPT // CH
NODE_CHUNKS = N // CH
ZR = 80
ECH = 128
EPTP = 10240
ENCHUNK = EPTP // ECH
EP = NW * EPTP
NP = N + ZR
PCH = 8
NPAN = ENCHUNK // PCH
DEGW = 640
DEGP = NS * DEGW

_dot = functools.partial(
    jnp.dot, precision=lax.Precision.DEFAULT, preferred_element_type=jnp.float32
)
_dot_x = functools.partial(
    jnp.dot, precision=lax.Precision.HIGHEST, preferred_element_type=jnp.float32
)


def _sc_embed_body(x_h, dst_h, ntab_h,
                   nf_h, deg0_h, deg1_h,
                   xv, dv, nfv, ones_v, zb, deg_sh, sem):
    c = lax.axis_index("c")
    s = lax.axis_index("s")
    w = c * NS + s

    def _fill_zb(i, carry):
        zb[pl.ds(i * 16, 16)] = jnp.zeros((16,), jnp.float32)
        return carry
    lax.fori_loop(0, DEGW // 16, _fill_zb, 0)

    def _fill_ones(i, carry):
        ones_v[pl.ds(i * 16, 16)] = jnp.ones((16,), jnp.float32)
        return carry
    lax.fori_loop(0, CH // 16, _fill_ones, 0)

    pltpu.sync_copy(zb, deg_sh.at[pl.ds(s * DEGW, DEGW)])
    plsc.subcore_barrier()

    def _deg(i, carry):
        base = w * EPT + i * CH
        pltpu.sync_copy(dst_h.at[pl.ds(base, CH)], dv)
        pltpu.sync_copy(ones_v, deg_sh.at[dv], add=True)
        return carry
    lax.fori_loop(0, NCHUNK, _deg, 0)

    def _emb(i, carry):
        j = w + NW * i

        @pl.when(j < NODE_CHUNKS)
        def _():
            base = j * CH
            pltpu.sync_copy(x_h.at[pl.ds(base, CH)], xv)
            pltpu.async_copy(ntab_h.at[xv], nfv, sem).wait()
            pltpu.sync_copy(nfv, nf_h.at[pl.ds(base, CH)])
        return carry
    lax.fori_loop(0, (NODE_CHUNKS + NW - 1) // NW, _emb, 0)

    plsc.subcore_barrier()

    @pl.when(c == 0)
    def _():
        pltpu.sync_copy(deg_sh.at[pl.ds(s * DEGW, DEGW)],
                        deg0_h.at[pl.ds(s * DEGW, DEGW)])

    @pl.when(c == 1)
    def _():
        pltpu.sync_copy(deg_sh.at[pl.ds(s * DEGW, DEGW)],
                        deg1_h.at[pl.ds(s * DEGW, DEGW)])


@functools.cache
def _get_sc_embed():
    return pl.kernel(
        _sc_embed_body,
        out_type=[
            jax.ShapeDtypeStruct((N, ED), jnp.float32),
            jax.ShapeDtypeStruct((DEGP,), jnp.float32),
            jax.ShapeDtypeStruct((DEGP,), jnp.float32),
        ],
        scratch_types=[
            pltpu.VMEM((CH,), jnp.int32),
            pltpu.VMEM((CH,), jnp.int32),
            pltpu.VMEM((CH, ED), jnp.float32),
            pltpu.VMEM((CH,), jnp.float32),
            pltpu.VMEM((DEGW,), jnp.float32),
            pltpu.VMEM_SHARED((DEGP,), jnp.float32),
            pltpu.SemaphoreType.DMA,
        ],
        mesh=plsc.VectorSubcoreMesh(core_axis_name="c", subcore_axis_name="s"),
    )


def _panel(hs_h, acc_sh, sv, dv, rows0, rows1, gsem0, gsem1):
    pltpu.async_copy(hs_h.at[sv.at[0]], rows0, gsem0)
    for j in range(PCH // 2):
        c0 = 2 * j
        c1 = c0 + 1
        pltpu.async_copy(hs_h.at[sv.at[c1]], rows1, gsem1)
        pltpu.make_async_copy(hs_h.at[sv.at[c0]], rows0, gsem0).wait()
        pltpu.sync_copy(rows0, acc_sh.at[dv.at[c0]], add=True)
        if c1 + 1 < PCH:
            pltpu.async_copy(hs_h.at[sv.at[c1 + 1]], rows0, gsem0)
        pltpu.make_async_copy(hs_h.at[sv.at[c1]], rows1, gsem1).wait()
        pltpu.sync_copy(rows1, acc_sh.at[dv.at[c1]], add=True)


def _sc_scatter_body(src_h, dst_h, hs_h, out_h,
                     svA, dvA, svB, dvB, rows0, rows1, acc_sh,
                     gsem0, gsem1, isemA, isemB):
    c = lax.axis_index("c")
    s = lax.axis_index("s")
    w = c * NS + s
    base = w * ENCHUNK

    def _zrow(r, carry):
        def _zcol(k, inner):
            rows0[r, pl.ds(k * 16, 16)] = jnp.zeros((16,), jnp.float32)
            return inner
        return lax.fori_loop(0, H // 16, _zcol, carry)
    lax.fori_loop(0, ECH, _zrow, 0)

    def _zacc(t, carry):
        j = s + NS * t

        @pl.when(j < NP // ZR)
        def _():
            pltpu.sync_copy(rows0.at[pl.ds(0, ZR)], acc_sh.at[pl.ds(j * ZR, ZR)])
        return carry
    lax.fori_loop(0, (NP // ZR + NS - 1) // NS, _zacc, 0)
    plsc.subcore_barrier()

    pltpu.sync_copy(src_h.at[pl.ds(base, PCH)], svA)
    pltpu.sync_copy(dst_h.at[pl.ds(base, PCH)], dvA)

    def _pp(pp, carry):
        pa = base + 2 * pp * PCH
        pltpu.async_copy(src_h.at[pl.ds(pa + PCH, PCH)], svB, isemB)
        pltpu.async_copy(dst_h.at[pl.ds(pa + PCH, PCH)], dvB, isemB)
        _panel(hs_h, acc_sh, svA, dvA, rows0, rows1, gsem0, gsem1)
        pltpu.make_async_copy(src_h.at[pl.ds(pa + PCH, PCH)], svB, isemB).wait()
        pltpu.make_async_copy(dst_h.at[pl.ds(pa + PCH, PCH)], dvB, isemB).wait()

        @pl.when(pp < NPAN // 2 - 1)
        def _():
            pltpu.async_copy(src_h.at[pl.ds(pa + 2 * PCH, PCH)], svA, isemA)
            pltpu.async_copy(dst_h.at[pl.ds(pa + 2 * PCH, PCH)], dvA, isemA)
        _panel(hs_h, acc_sh, svB, dvB, rows0, rows1, gsem0, gsem1)

        @pl.when(pp < NPAN // 2 - 1)
        def _():
            pltpu.make_async_copy(src_h.at[pl.ds(pa + 2 * PCH, PCH)], svA,
                                  isemA).wait()
            pltpu.make_async_copy(dst_h.at[pl.ds(pa + 2 * PCH, PCH)], dvA,
                                  isemA).wait()
        return carry
    lax.fori_loop(0, NPAN // 2, _pp, 0)

    plsc.subcore_barrier()

    def _drain(t, carry):
        j = s + NS * t

        @pl.when(j < N // ZR)
        def _():
            pltpu.sync_copy(acc_sh.at[pl.ds(j * ZR, ZR)],
                            out_h.at[c, pl.ds(j * ZR, ZR)])
        return carry
    lax.fori_loop(0, (N // ZR + NS - 1) // NS, _drain, 0)


@functools.cache
def _get_sc_scatter():
    return pl.kernel(
        _sc_scatter_body,
        out_type=[jax.ShapeDtypeStruct((NC, N, H), jnp.float32)],
        scratch_types=[
            pltpu.VMEM((PCH, ECH), jnp.int32),
            pltpu.VMEM((PCH, ECH), jnp.int32),
            pltpu.VMEM((PCH, ECH), jnp.int32),
            pltpu.VMEM((PCH, ECH), jnp.int32),
            pltpu.VMEM((ECH, H), jnp.float32),
            pltpu.VMEM((ECH, H), jnp.float32),
            pltpu.VMEM_SHARED((NP, H), jnp.float32),
            pltpu.SemaphoreType.DMA,
            pltpu.SemaphoreType.DMA,
            pltpu.SemaphoreType.DMA,
            pltpu.SemaphoreType.DMA,
        ],
        mesh=plsc.VectorSubcoreMesh(core_axis_name="c", subcore_axis_name="s"),
    )


R = 1000


def _tc_proj_body(nf, ndb, cib, degp, dtab, ctab, pw, pb, w1, hs_out):
    oh_d = (lax.broadcasted_iota(jnp.int32, (R, DD1), 1) == ndb[...]).astype(
        jnp.float32)
    oh_c = (lax.broadcasted_iota(jnp.int32, (R, CD1), 1) == cib[...]).astype(
        jnp.float32)
    df = _dot_x(oh_d, dtab[...])
    cf = _dot_x(oh_c, ctab[...])
    cat = jnp.concatenate([nf[...], df, cf], axis=1)
    h = jnp.maximum(_dot(cat, pw[...]) + pb[...], 0.0)
    dinv = lax.rsqrt(1.0 + degp[:, 0] + degp[:, 1])
    hs_out[...] = _dot(h, w1[...]) * dinv[:, None]


def _tc_mid_body(p0, p1, hs1, degp, b1, w2, hs2_out):
    dinv = lax.rsqrt(1.0 + degp[:, 0] + degp[:, 1])[:, None]
    h2 = jnp.maximum((p0[...] + p1[...] + hs1[...]) * dinv + b1[...], 0.0)
    hs2_out[...] = _dot(h2, w2[...]) * dinv


def _tc_final_body(q0, q1, hs2, degp, b2, batch, cw, cb, out):
    dinv = lax.rsqrt(1.0 + degp[:, 0] + degp[:, 1])[:, None]
    h3 = jnp.maximum((q0[...] + q1[...] + hs2[...]) * dinv + b2[...], 0.0)
    seg = lax.broadcasted_iota(jnp.int32, (NG, N), 0)
    m = (seg == batch[...]).astype(jnp.float32)
    sums = _dot_x(m, h3)
    counts = jnp.sum(m, axis=1, keepdims=True)
    pooled = sums / jnp.maximum(counts, 1.0)
    out[...] = _dot(pooled, cw[...]) + cb[...]


def _row_spec(cols):
    return pl.BlockSpec((R, cols), lambda j: (j, 0))


def _bcast_spec(rows, cols):
    return pl.BlockSpec((rows, cols), lambda j: (0, 0))


_deg_spec = pl.BlockSpec((R, NC), lambda j: (j, 0))

_tc_proj = pl.pallas_call(
    _tc_proj_body,
    grid=(N // R,),
    in_specs=[
        _row_spec(ED), _row_spec(1), _row_spec(1), _deg_spec,
        _bcast_spec(DD1, DD), _bcast_spec(CD1, CD),
        _bcast_spec(ED + DD + CD, H),
        _bcast_spec(1, H), _bcast_spec(H, H),
    ],
    out_specs=_row_spec(H),
    out_shape=jax.ShapeDtypeStruct((N, H), jnp.float32),
)

_tc_mid = pl.pallas_call(
    _tc_mid_body,
    grid=(N // R,),
    in_specs=[
        _row_spec(H), _row_spec(H), _row_spec(H), _deg_spec,
        _bcast_spec(1, H), _bcast_spec(H, H),
    ],
    out_specs=_row_spec(H),
    out_shape=jax.ShapeDtypeStruct((N, H), jnp.float32),
)

_tc_final = pl.pallas_call(
    _tc_final_body,
    out_shape=jax.ShapeDtypeStruct((NG, 1), jnp.float32),
)


def kernel(x, edge_index, batch, node_depth, child_index, node_table,
           depth_table, child_table, proj_W, proj_b, conv1_W, conv1_b,
           conv2_W, conv2_b, clf_W, clf_b):
    src = edge_index[0]
    dst = edge_index[1]

    nf, deg0, deg1 = _get_sc_embed()(
        x.astype(jnp.int32), dst.astype(jnp.int32), node_table)
    degp = jnp.stack([deg0[:N], deg1[:N]], axis=1)

    hs1 = _tc_proj(nf, node_depth.astype(jnp.int32)[:, None],
                   child_index.astype(jnp.int32)[:, None], degp,
                   depth_table, child_table, proj_W, proj_b[None, :],
                   conv1_W)
    pad = EP - E
    srcp = jnp.concatenate(
        [src.astype(jnp.int32), jnp.zeros((pad,), jnp.int32)]
    ).reshape(NW * ENCHUNK, ECH)
    dstp = jnp.concatenate(
        [dst.astype(jnp.int32), N + (jnp.arange(pad, dtype=jnp.int32) % ZR)]
    ).reshape(NW * ENCHUNK, ECH)
    scatter = _get_sc_scatter()
    p = scatter(srcp, dstp, hs1)[0]
    hs2 = _tc_mid(p[0], p[1], hs1, degp, conv1_b[None, :], conv2_W)
    q = scatter(srcp, dstp, hs2)[0]
    return _tc_final(q[0], q[1], hs2, degp, conv2_b[None, :],
                     batch[None, :].astype(jnp.int32), clf_W, clf_b[None, :])


# speedup vs baseline: 2.8123x; 2.6805x over previous
import functools

import jax
import jax.numpy as jnp
from jax import lax
from jax.experimental import pallas as pl
from jax.experimental.pallas import tpu as pltpu
from jax.experimental.pallas import tpu_sc as plsc

N = 10000
E = 320000
NG = 64
ED = 128
DD = 32
CD = 32
H = 128
DD1 = 51
CD1 = 21

NC = 2
NS = 16
NW = NC * NS

EPT = E // NW
CH = 80
NCHUNK = EPT // CH
NODE_CHUNKS = N // CH
ZR = 80
ECH = 128
EPTP = 10240
ENCHUNK = EPTP // ECH
EP = NW * EPTP
NP = N + ZR
PCH = 8
NPAN = ENCHUNK // PCH
DEGW = 640
DEGP = NS * DEGW

_dot = functools.partial(
    jnp.dot, precision=lax.Precision.DEFAULT, preferred_element_type=jnp.float32
)
_dot_x = functools.partial(
    jnp.dot, precision=lax.Precision.HIGHEST, preferred_element_type=jnp.float32
)


def _sc_embed_body(x_h, dst_h, ntab_h,
                   nf_h, deg0_h, deg1_h,
                   xv, dv, nfv, ones_v, zb, deg_sh, sem):
    c = lax.axis_index("c")
    s = lax.axis_index("s")
    w = c * NS + s

    def _fill_zb(i, carry):
        zb[pl.ds(i * 16, 16)] = jnp.zeros((16,), jnp.float32)
        return carry
    lax.fori_loop(0, DEGW // 16, _fill_zb, 0)

    def _fill_ones(i, carry):
        ones_v[pl.ds(i * 16, 16)] = jnp.ones((16,), jnp.float32)
        return carry
    lax.fori_loop(0, CH // 16, _fill_ones, 0)

    pltpu.sync_copy(zb, deg_sh.at[pl.ds(s * DEGW, DEGW)])
    plsc.subcore_barrier()

    def _deg(i, carry):
        base = w * EPT + i * CH
        pltpu.sync_copy(dst_h.at[pl.ds(base, CH)], dv)
        pltpu.sync_copy(ones_v, deg_sh.at[dv], add=True)
        return carry
    lax.fori_loop(0, NCHUNK, _deg, 0)

    def _emb(i, carry):
        j = w + NW * i

        @pl.when(j < NODE_CHUNKS)
        def _():
            base = j * CH
            pltpu.sync_copy(x_h.at[pl.ds(base, CH)], xv)
            pltpu.async_copy(ntab_h.at[xv], nfv, sem).wait()
            pltpu.sync_copy(nfv, nf_h.at[pl.ds(base, CH)])
        return carry
    lax.fori_loop(0, (NODE_CHUNKS + NW - 1) // NW, _emb, 0)

    plsc.subcore_barrier()

    @pl.when(c == 0)
    def _():
        pltpu.sync_copy(deg_sh.at[pl.ds(s * DEGW, DEGW)],
                        deg0_h.at[pl.ds(s * DEGW, DEGW)])

    @pl.when(c == 1)
    def _():
        pltpu.sync_copy(deg_sh.at[pl.ds(s * DEGW, DEGW)],
                        deg1_h.at[pl.ds(s * DEGW, DEGW)])


@functools.cache
def _get_sc_embed():
    return pl.kernel(
        _sc_embed_body,
        out_type=[
            jax.ShapeDtypeStruct((N, ED), jnp.float32),
            jax.ShapeDtypeStruct((DEGP,), jnp.float32),
            jax.ShapeDtypeStruct((DEGP,), jnp.float32),
        ],
        scratch_types=[
            pltpu.VMEM((CH,), jnp.int32),
            pltpu.VMEM((CH,), jnp.int32),
            pltpu.VMEM((CH, ED), jnp.float32),
            pltpu.VMEM((CH,), jnp.float32),
            pltpu.VMEM((DEGW,), jnp.float32),
            pltpu.VMEM_SHARED((DEGP,), jnp.float32),
            pltpu.SemaphoreType.DMA,
        ],
        mesh=plsc.VectorSubcoreMesh(core_axis_name="c", subcore_axis_name="s"),
    )


def _panel(hs_h, acc_sh, sv, dv, rows0, rows1, gsem0, gsem1):
    pltpu.async_copy(hs_h.at[sv.at[0]], rows0, gsem0)
    for j in range(PCH // 2):
        c0 = 2 * j
        c1 = c0 + 1
        pltpu.async_copy(hs_h.at[sv.at[c1]], rows1, gsem1)
        pltpu.make_async_copy(hs_h.at[sv.at[c0]], rows0, gsem0).wait()
        pltpu.sync_copy(rows0, acc_sh.at[dv.at[c0]], add=True)
        if c1 + 1 < PCH:
            pltpu.async_copy(hs_h.at[sv.at[c1 + 1]], rows0, gsem0)
        pltpu.make_async_copy(hs_h.at[sv.at[c1]], rows1, gsem1).wait()
        pltpu.sync_copy(rows1, acc_sh.at[dv.at[c1]], add=True)


def _sc_scatter_body(src_h, dst_h, hs_h, out_h,
                     svA, dvA, svB, dvB, rows0, rows1, acc_sh,
                     gsem0, gsem1, isemA, isemB):
    c = lax.axis_index("c")
    s = lax.axis_index("s")
    w = c * NS + s
    base = w * ENCHUNK

    def _zrow(r, carry):
        def _zcol(k, inner):
            rows0[r, pl.ds(k * 16, 16)] = jnp.zeros((16,), jnp.float32)
            return inner
        return lax.fori_loop(0, H // 16, _zcol, carry)
    lax.fori_loop(0, ECH, _zrow, 0)

    def _zacc(t, carry):
        j = s + NS * t

        @pl.when(j < NP // ZR)
        def _():
            pltpu.sync_copy(rows0.at[pl.ds(0, ZR)], acc_sh.at[pl.ds(j * ZR, ZR)])
        return carry
    lax.fori_loop(0, (NP // ZR + NS - 1) // NS, _zacc, 0)
    plsc.subcore_barrier()

    pltpu.sync_copy(src_h.at[pl.ds(base, PCH)], svA)
    pltpu.sync_copy(dst_h.at[pl.ds(base, PCH)], dvA)

    def _pp(pp, carry):
        pa = base + 2 * pp * PCH
        pltpu.async_copy(src_h.at[pl.ds(pa + PCH, PCH)], svB, isemB)
        pltpu.async_copy(dst_h.at[pl.ds(pa + PCH, PCH)], dvB, isemB)
        _panel(hs_h, acc_sh, svA, dvA, rows0, rows1, gsem0, gsem1)
        pltpu.make_async_copy(src_h.at[pl.ds(pa + PCH, PCH)], svB, isemB).wait()
        pltpu.make_async_copy(dst_h.at[pl.ds(pa + PCH, PCH)], dvB, isemB).wait()

        @pl.when(pp < NPAN // 2 - 1)
        def _():
            pltpu.async_copy(src_h.at[pl.ds(pa + 2 * PCH, PCH)], svA, isemA)
            pltpu.async_copy(dst_h.at[pl.ds(pa + 2 * PCH, PCH)], dvA, isemA)
        _panel(hs_h, acc_sh, svB, dvB, rows0, rows1, gsem0, gsem1)

        @pl.when(pp < NPAN // 2 - 1)
        def _():
            pltpu.make_async_copy(src_h.at[pl.ds(pa + 2 * PCH, PCH)], svA,
                                  isemA).wait()
            pltpu.make_async_copy(dst_h.at[pl.ds(pa + 2 * PCH, PCH)], dvA,
                                  isemA).wait()
        return carry
    lax.fori_loop(0, NPAN // 2, _pp, 0)

    plsc.subcore_barrier()

    def _drain(t, carry):
        j = s + NS * t

        @pl.when(j < N // ZR)
        def _():
            pltpu.sync_copy(acc_sh.at[pl.ds(j * ZR, ZR)],
                            out_h.at[c, pl.ds(j * ZR, ZR)])
        return carry
    lax.fori_loop(0, (N // ZR + NS - 1) // NS, _drain, 0)


@functools.cache
def _get_sc_scatter():
    return pl.kernel(
        _sc_scatter_body,
        out_type=[jax.ShapeDtypeStruct((NC, N, H), jnp.float32)],
        scratch_types=[
            pltpu.VMEM((PCH, ECH), jnp.int32),
            pltpu.VMEM((PCH, ECH), jnp.int32),
            pltpu.VMEM((PCH, ECH), jnp.int32),
            pltpu.VMEM((PCH, ECH), jnp.int32),
            pltpu.VMEM((ECH, H), jnp.float32),
            pltpu.VMEM((ECH, H), jnp.float32),
            pltpu.VMEM_SHARED((NP, H), jnp.float32),
            pltpu.SemaphoreType.DMA,
            pltpu.SemaphoreType.DMA,
            pltpu.SemaphoreType.DMA,
            pltpu.SemaphoreType.DMA,
        ],
        mesh=plsc.VectorSubcoreMesh(core_axis_name="c", subcore_axis_name="s"),
    )


R = 1000


def _tc_proj_body(nf, ndb, cib, degp, dtab, ctab, pw, pb, w1, hs_out):
    oh_d = (lax.broadcasted_iota(jnp.int32, (R, DD1), 1) == ndb[...]).astype(
        jnp.float32)
    oh_c = (lax.broadcasted_iota(jnp.int32, (R, CD1), 1) == cib[...]).astype(
        jnp.float32)
    df = _dot_x(oh_d, dtab[...])
    cf = _dot_x(oh_c, ctab[...])
    cat = jnp.concatenate([nf[...], df, cf], axis=1)
    h = jnp.maximum(_dot(cat, pw[...]) + pb[...], 0.0)
    dinv = lax.rsqrt(1.0 + degp[:, 0] + degp[:, 1])
    hs_out[...] = _dot(h, w1[...]) * dinv[:, None]


def _tc_mid_body(p0, p1, hs1, degp, b1, w2, hs2_out):
    dinv = lax.rsqrt(1.0 + degp[:, 0] + degp[:, 1])[:, None]
    h2 = jnp.maximum((p0[...] + p1[...] + hs1[...]) * dinv + b1[...], 0.0)
    hs2_out[...] = _dot(h2, w2[...]) * dinv


def _tc_final_body(q0, q1, hs2, degp, b2, batch, cw, cb, out):
    dinv = lax.rsqrt(1.0 + degp[:, 0] + degp[:, 1])[:, None]
    h3 = jnp.maximum((q0[...] + q1[...] + hs2[...]) * dinv + b2[...], 0.0)
    seg = lax.broadcasted_iota(jnp.int32, (NG, N), 0)
    m = (seg == batch[...]).astype(jnp.float32)
    sums = _dot_x(m, h3)
    counts = jnp.sum(m, axis=1, keepdims=True)
    pooled = sums / jnp.maximum(counts, 1.0)
    out[...] = _dot(pooled, cw[...]) + cb[...]


def _row_spec(cols):
    return pl.BlockSpec((R, cols), lambda j: (j, 0))


def _bcast_spec(rows, cols):
    return pl.BlockSpec((rows, cols), lambda j: (0, 0))


_deg_spec = pl.BlockSpec((R, NC), lambda j: (j, 0))

_tc_proj = pl.pallas_call(
    _tc_proj_body,
    grid=(N // R,),
    in_specs=[
        _row_spec(ED), _row_spec(1), _row_spec(1), _deg_spec,
        _bcast_spec(DD1, DD), _bcast_spec(CD1, CD),
        _bcast_spec(ED + DD + CD, H),
        _bcast_spec(1, H), _bcast_spec(H, H),
    ],
    out_specs=_row_spec(H),
    out_shape=jax.ShapeDtypeStruct((N, H), jnp.float32),
)

_tc_mid = pl.pallas_call(
    _tc_mid_body,
    grid=(N // R,),
    in_specs=[
        _row_spec(H), _row_spec(H), _row_spec(H), _deg_spec,
        _bcast_spec(1, H), _bcast_spec(H, H),
    ],
    out_specs=_row_spec(H),
    out_shape=jax.ShapeDtypeStruct((N, H), jnp.float32),
)

_tc_final = pl.pallas_call(
    _tc_final_body,
    out_shape=jax.ShapeDtypeStruct((NG, 1), jnp.float32),
)


def kernel(x, edge_index, batch, node_depth, child_index, node_table,
           depth_table, child_table, proj_W, proj_b, conv1_W, conv1_b,
           conv2_W, conv2_b, clf_W, clf_b):
    src = edge_index[0]
    dst = edge_index[1]

    nf, deg0, deg1 = _get_sc_embed()(
        x.astype(jnp.int32), dst.astype(jnp.int32), node_table)
    degp = jnp.stack([deg0[:N], deg1[:N]], axis=1)

    hs1 = _tc_proj(nf, node_depth.astype(jnp.int32)[:, None],
                   child_index.astype(jnp.int32)[:, None], degp,
                   depth_table, child_table, proj_W, proj_b[None, :],
                   conv1_W)
    ppt = EPTP - EPT
    pad_s = (jnp.arange(NW * ppt, dtype=jnp.int32) % N).reshape(NW, ppt)
    pad_d = N + (jnp.arange(NW * ppt, dtype=jnp.int32) % ZR).reshape(NW, ppt)
    srcp = jnp.concatenate(
        [src.astype(jnp.int32).reshape(NW, EPT), pad_s], axis=1
    ).reshape(NW * ENCHUNK, ECH)
    dstp = jnp.concatenate(
        [dst.astype(jnp.int32).reshape(NW, EPT), pad_d], axis=1
    ).reshape(NW * ENCHUNK, ECH)
    scatter = _get_sc_scatter()
    p = scatter(srcp, dstp, hs1)[0]
    hs2 = _tc_mid(p[0], p[1], hs1, degp, conv1_b[None, :], conv2_W)
    q = scatter(srcp, dstp, hs2)[0]
    return _tc_final(q[0], q[1], hs2, degp, conv2_b[None, :],
                     batch[None, :].astype(jnp.int32), clf_W, clf_b[None, :])


# speedup vs baseline: 3.2458x; 1.1541x over previous
import functools

import jax
import jax.numpy as jnp
from jax import lax
from jax.experimental import pallas as pl
from jax.experimental.pallas import tpu as pltpu
from jax.experimental.pallas import tpu_sc as plsc

N = 10000
E = 320000
NG = 64
ED = 128
DD = 32
CD = 32
H = 128
DD1 = 51
CD1 = 21

NC = 2
NS = 16
NW = NC * NS

EPT = E // NW
CH = 80
NCHUNK = EPT // CH
NODE_CHUNKS = N // CH
ZR = 80
ECH = 128
EPTP = 10240
ENCHUNK = EPTP // ECH
EP = NW * EPTP
NP = N + ZR
PCH = 8
NPAN = ENCHUNK // PCH
DEGW = 640
DEGP = NS * DEGW

_dot = functools.partial(
    jnp.dot, precision=lax.Precision.DEFAULT, preferred_element_type=jnp.float32
)
_dot_x = functools.partial(
    jnp.dot, precision=lax.Precision.HIGHEST, preferred_element_type=jnp.float32
)


def _sc_embed_body(x_h, dst_h, ntab_h,
                   nf_h, deg0_h, deg1_h,
                   xv0, xv1, nfv0, nfv1, dvA, dvB, ones_v, zb, deg_sh,
                   gsem0, gsem1, isemA, isemB):
    c = lax.axis_index("c")
    s = lax.axis_index("s")
    w = c * NS + s
    base = w * ENCHUNK

    def _fill_zb(i, carry):
        zb[pl.ds(i * 16, 16)] = jnp.zeros((16,), jnp.float32)
        return carry
    lax.fori_loop(0, DEGW // 16, _fill_zb, 0)

    def _fill_ones(i, carry):
        ones_v[pl.ds(i * 16, 16)] = jnp.ones((16,), jnp.float32)
        return carry
    lax.fori_loop(0, ECH // 16, _fill_ones, 0)

    pltpu.sync_copy(zb, deg_sh.at[pl.ds(s * DEGW, DEGW)])
    plsc.subcore_barrier()

    def _emb(i, carry):
        j = w + NW * i

        @pl.when(j < NODE_CHUNKS)
        def _():
            pltpu.sync_copy(x_h.at[pl.ds(j * CH, CH)], xv0)
            pltpu.async_copy(ntab_h.at[xv0], nfv0, gsem0).wait()
            pltpu.sync_copy(nfv0, nf_h.at[pl.ds(j * CH, CH)])
        return carry
    lax.fori_loop(0, (NODE_CHUNKS + NW - 1) // NW, _emb, 0)

    pltpu.sync_copy(dst_h.at[pl.ds(base, PCH)], dvA)

    def _dpanel(dv):
        for j in range(PCH):
            pltpu.sync_copy(ones_v, deg_sh.at[dv.at[j]], add=True)

    def _pp(pp, carry):
        pa = base + 2 * pp * PCH
        pltpu.async_copy(dst_h.at[pl.ds(pa + PCH, PCH)], dvB, isemB)
        _dpanel(dvA)
        pltpu.make_async_copy(dst_h.at[pl.ds(pa + PCH, PCH)], dvB, isemB).wait()

        @pl.when(pp < NPAN // 2 - 1)
        def _():
            pltpu.async_copy(dst_h.at[pl.ds(pa + 2 * PCH, PCH)], dvA, isemA)
        _dpanel(dvB)

        @pl.when(pp < NPAN // 2 - 1)
        def _():
            pltpu.make_async_copy(dst_h.at[pl.ds(pa + 2 * PCH, PCH)], dvA,
                                  isemA).wait()
        return carry
    lax.fori_loop(0, NPAN // 2, _pp, 0)

    plsc.subcore_barrier()

    @pl.when(c == 0)
    def _():
        pltpu.sync_copy(deg_sh.at[pl.ds(s * DEGW, DEGW)],
                        deg0_h.at[pl.ds(s * DEGW, DEGW)])

    @pl.when(c == 1)
    def _():
        pltpu.sync_copy(deg_sh.at[pl.ds(s * DEGW, DEGW)],
                        deg1_h.at[pl.ds(s * DEGW, DEGW)])


@functools.cache
def _get_sc_embed():
    return pl.kernel(
        _sc_embed_body,
        out_type=[
            jax.ShapeDtypeStruct((N, ED), jnp.float32),
            jax.ShapeDtypeStruct((DEGP,), jnp.float32),
            jax.ShapeDtypeStruct((DEGP,), jnp.float32),
        ],
        scratch_types=[
            pltpu.VMEM((CH,), jnp.int32),
            pltpu.VMEM((CH,), jnp.int32),
            pltpu.VMEM((CH, ED), jnp.float32),
            pltpu.VMEM((CH, ED), jnp.float32),
            pltpu.VMEM((PCH, ECH), jnp.int32),
            pltpu.VMEM((PCH, ECH), jnp.int32),
            pltpu.VMEM((ECH,), jnp.float32),
            pltpu.VMEM((DEGW,), jnp.float32),
            pltpu.VMEM_SHARED((DEGP,), jnp.float32),
            pltpu.SemaphoreType.DMA,
            pltpu.SemaphoreType.DMA,
            pltpu.SemaphoreType.DMA,
            pltpu.SemaphoreType.DMA,
        ],
        mesh=plsc.VectorSubcoreMesh(core_axis_name="c", subcore_axis_name="s"),
    )


def _panel(hs_h, acc_sh, sv, dv, rows0, rows1, gsem0, gsem1):
    pltpu.async_copy(hs_h.at[sv.at[0]], rows0, gsem0)
    for j in range(PCH // 2):
        c0 = 2 * j
        c1 = c0 + 1
        pltpu.async_copy(hs_h.at[sv.at[c1]], rows1, gsem1)
        pltpu.make_async_copy(hs_h.at[sv.at[c0]], rows0, gsem0).wait()
        pltpu.sync_copy(rows0, acc_sh.at[dv.at[c0]], add=True)
        if c1 + 1 < PCH:
            pltpu.async_copy(hs_h.at[sv.at[c1 + 1]], rows0, gsem0)
        pltpu.make_async_copy(hs_h.at[sv.at[c1]], rows1, gsem1).wait()
        pltpu.sync_copy(rows1, acc_sh.at[dv.at[c1]], add=True)


def _sc_scatter_body(src_h, dst_h, hs_h, out_h,
                     svA, dvA, svB, dvB, rows0, rows1, acc_sh,
                     gsem0, gsem1, isemA, isemB):
    c = lax.axis_index("c")
    s = lax.axis_index("s")
    w = c * NS + s
    base = w * ENCHUNK

    def _zrow(r, carry):
        def _zcol(k, inner):
            rows0[r, pl.ds(k * 16, 16)] = jnp.zeros((16,), jnp.float32)
            return inner
        return lax.fori_loop(0, H // 16, _zcol, carry)
    lax.fori_loop(0, ECH, _zrow, 0)

    def _zacc(t, carry):
        j = s + NS * t

        @pl.when(j < NP // ZR)
        def _():
            pltpu.sync_copy(rows0.at[pl.ds(0, ZR)], acc_sh.at[pl.ds(j * ZR, ZR)])
        return carry
    lax.fori_loop(0, (NP // ZR + NS - 1) // NS, _zacc, 0)
    plsc.subcore_barrier()

    pltpu.sync_copy(src_h.at[pl.ds(base, PCH)], svA)
    pltpu.sync_copy(dst_h.at[pl.ds(base, PCH)], dvA)

    def _pp(pp, carry):
        pa = base + 2 * pp * PCH
        pltpu.async_copy(src_h.at[pl.ds(pa + PCH, PCH)], svB, isemB)
        pltpu.async_copy(dst_h.at[pl.ds(pa + PCH, PCH)], dvB, isemB)
        _panel(hs_h, acc_sh, svA, dvA, rows0, rows1, gsem0, gsem1)
        pltpu.make_async_copy(src_h.at[pl.ds(pa + PCH, PCH)], svB, isemB).wait()
        pltpu.make_async_copy(dst_h.at[pl.ds(pa + PCH, PCH)], dvB, isemB).wait()

        @pl.when(pp < NPAN // 2 - 1)
        def _():
            pltpu.async_copy(src_h.at[pl.ds(pa + 2 * PCH, PCH)], svA, isemA)
            pltpu.async_copy(dst_h.at[pl.ds(pa + 2 * PCH, PCH)], dvA, isemA)
        _panel(hs_h, acc_sh, svB, dvB, rows0, rows1, gsem0, gsem1)

        @pl.when(pp < NPAN // 2 - 1)
        def _():
            pltpu.make_async_copy(src_h.at[pl.ds(pa + 2 * PCH, PCH)], svA,
                                  isemA).wait()
            pltpu.make_async_copy(dst_h.at[pl.ds(pa + 2 * PCH, PCH)], dvA,
                                  isemA).wait()
        return carry
    lax.fori_loop(0, NPAN // 2, _pp, 0)

    plsc.subcore_barrier()

    def _drain(t, carry):
        j = s + NS * t

        @pl.when(j < N // ZR)
        def _():
            pltpu.sync_copy(acc_sh.at[pl.ds(j * ZR, ZR)],
                            out_h.at[c, pl.ds(j * ZR, ZR)])
        return carry
    lax.fori_loop(0, (N // ZR + NS - 1) // NS, _drain, 0)


@functools.cache
def _get_sc_scatter():
    return pl.kernel(
        _sc_scatter_body,
        out_type=[jax.ShapeDtypeStruct((NC, N, H), jnp.float32)],
        scratch_types=[
            pltpu.VMEM((PCH, ECH), jnp.int32),
            pltpu.VMEM((PCH, ECH), jnp.int32),
            pltpu.VMEM((PCH, ECH), jnp.int32),
            pltpu.VMEM((PCH, ECH), jnp.int32),
            pltpu.VMEM((ECH, H), jnp.float32),
            pltpu.VMEM((ECH, H), jnp.float32),
            pltpu.VMEM_SHARED((NP, H), jnp.float32),
            pltpu.SemaphoreType.DMA,
            pltpu.SemaphoreType.DMA,
            pltpu.SemaphoreType.DMA,
            pltpu.SemaphoreType.DMA,
        ],
        mesh=plsc.VectorSubcoreMesh(core_axis_name="c", subcore_axis_name="s"),
    )


R = 1000


def _tc_proj_body(nf, ndb, cib, degp, dtab, ctab, pw, pb, w1, hs_out):
    oh_d = (lax.broadcasted_iota(jnp.int32, (R, DD1), 1) == ndb[...]).astype(
        jnp.float32)
    oh_c = (lax.broadcasted_iota(jnp.int32, (R, CD1), 1) == cib[...]).astype(
        jnp.float32)
    df = _dot_x(oh_d, dtab[...])
    cf = _dot_x(oh_c, ctab[...])
    cat = jnp.concatenate([nf[...], df, cf], axis=1)
    h = jnp.maximum(_dot(cat, pw[...]) + pb[...], 0.0)
    dinv = lax.rsqrt(1.0 + degp[:, 0] + degp[:, 1])
    hs_out[...] = _dot(h, w1[...]) * dinv[:, None]


def _tc_mid_body(p0, p1, hs1, degp, b1, w2, hs2_out):
    dinv = lax.rsqrt(1.0 + degp[:, 0] + degp[:, 1])[:, None]
    h2 = jnp.maximum((p0[...] + p1[...] + hs1[...]) * dinv + b1[...], 0.0)
    hs2_out[...] = _dot(h2, w2[...]) * dinv


def _tc_final_body(q0, q1, hs2, degp, b2, batch, cw, cb, out):
    dinv = lax.rsqrt(1.0 + degp[:, 0] + degp[:, 1])[:, None]
    h3 = jnp.maximum((q0[...] + q1[...] + hs2[...]) * dinv + b2[...], 0.0)
    seg = lax.broadcasted_iota(jnp.int32, (NG, N), 0)
    m = (seg == batch[...]).astype(jnp.float32)
    sums = _dot_x(m, h3)
    counts = jnp.sum(m, axis=1, keepdims=True)
    pooled = sums / jnp.maximum(counts, 1.0)
    out[...] = _dot(pooled, cw[...]) + cb[...]


def _row_spec(cols):
    return pl.BlockSpec((R, cols), lambda j: (j, 0))


def _bcast_spec(rows, cols):
    return pl.BlockSpec((rows, cols), lambda j: (0, 0))


_deg_spec = pl.BlockSpec((R, NC), lambda j: (j, 0))

_tc_proj = pl.pallas_call(
    _tc_proj_body,
    grid=(N // R,),
    in_specs=[
        _row_spec(ED), _row_spec(1), _row_spec(1), _deg_spec,
        _bcast_spec(DD1, DD), _bcast_spec(CD1, CD),
        _bcast_spec(ED + DD + CD, H),
        _bcast_spec(1, H), _bcast_spec(H, H),
    ],
    out_specs=_row_spec(H),
    out_shape=jax.ShapeDtypeStruct((N, H), jnp.float32),
)

_tc_mid = pl.pallas_call(
    _tc_mid_body,
    grid=(N // R,),
    in_specs=[
        _row_spec(H), _row_spec(H), _row_spec(H), _deg_spec,
        _bcast_spec(1, H), _bcast_spec(H, H),
    ],
    out_specs=_row_spec(H),
    out_shape=jax.ShapeDtypeStruct((N, H), jnp.float32),
)

_tc_final = pl.pallas_call(
    _tc_final_body,
    out_shape=jax.ShapeDtypeStruct((NG, 1), jnp.float32),
)


def kernel(x, edge_index, batch, node_depth, child_index, node_table,
           depth_table, child_table, proj_W, proj_b, conv1_W, conv1_b,
           conv2_W, conv2_b, clf_W, clf_b):
    src = edge_index[0]
    dst = edge_index[1]

    ppt = EPTP - EPT
    pad_s = (jnp.arange(NW * ppt, dtype=jnp.int32) % N).reshape(NW, ppt)
    pad_d = N + (jnp.arange(NW * ppt, dtype=jnp.int32) % ZR).reshape(NW, ppt)
    srcp = jnp.concatenate(
        [src.astype(jnp.int32).reshape(NW, EPT), pad_s], axis=1
    ).reshape(NW * ENCHUNK, ECH)
    dstp = jnp.concatenate(
        [dst.astype(jnp.int32).reshape(NW, EPT), pad_d], axis=1
    ).reshape(NW * ENCHUNK, ECH)

    nf, deg0, deg1 = _get_sc_embed()(x.astype(jnp.int32), dstp, node_table)
    degp = jnp.stack([deg0[:N], deg1[:N]], axis=1)

    hs1 = _tc_proj(nf, node_depth.astype(jnp.int32)[:, None],
                   child_index.astype(jnp.int32)[:, None], degp,
                   depth_table, child_table, proj_W, proj_b[None, :],
                   conv1_W)
    scatter = _get_sc_scatter()
    p = scatter(srcp, dstp, hs1)[0]
    hs2 = _tc_mid(p[0], p[1], hs1, degp, conv1_b[None, :], conv2_W)
    q = scatter(srcp, dstp, hs2)[0]
    return _tc_final(q[0], q[1], hs2, degp, conv2_b[None, :],
                     batch[None, :].astype(jnp.int32), clf_W, clf_b[None, :])
